# baseline (speedup 1.0000x reference)
import functools

import jax
import jax.numpy as jnp
from jax import lax
from jax.experimental import pallas as pl
from jax.experimental.pallas import tpu as pltpu
from jax.experimental.pallas import tpu_sc as plsc

_BATCH = 16384
_EMBED_DIM = 64
_OUT_PAD = 128
_NUM_CORES = 2
_NUM_SUBCORES = 16
_NUM_WORKERS = _NUM_CORES * _NUM_SUBCORES
_CHUNK = 512
_N_FULL_CHUNKS = 1953
_TAIL_COL0 = _N_FULL_CHUNKS * _CHUNK
_TAIL_W = 64
_RING = 16
_LANES = 16


def _iota16():
    return lax.broadcasted_iota(jnp.int32, (_LANES,), 0)


def _scalar(x):
    return x[0] if getattr(x, "ndim", 0) else x


def _gather_body(idx_hbm, tableT_hbm, tail_hbm, out_hbm, midx, mpos, gbuf,
                 chunk_v, tail_v, scr_mv, scr_mp, rowbufs, sem, csem):
    wid = lax.axis_index("s") * _NUM_CORES + lax.axis_index("c")
    pltpu.sync_copy(idx_hbm, gbuf.at[pl.ds(0, _BATCH)])

    def scan_step(t, cnt):
        v = gbuf[pl.ds(t * _LANES, _LANES)]
        pos = _iota16() + t * _LANES
        mine = ((v >> 9) & (_NUM_WORKERS - 1)) == wid
        plsc.store_compressed(midx.at[pl.ds(cnt, _LANES)], v, mask=mine)
        plsc.store_compressed(mpos.at[pl.ds(cnt, _LANES)], pos, mask=mine)
        return cnt + _scalar(plsc.all_reduce_population_count(mine))

    cnt = lax.fori_loop(0, _BATCH // _LANES, scan_step, 0)
    midx[pl.ds(cnt, _LANES)] = jnp.full((_LANES,), jnp.int32(0x7F000000))
    nvec = (cnt + _LANES - 1) // _LANES

    def process_chunk(c, col0, chunk_ref, mc, swap=False):
        def vec_step(t, mc):
            mv = midx[pl.ds(t * _LANES, _LANES)]
            mp = mpos[pl.ds(t * _LANES, _LANES)]
            hit = (mv >> 9) == c
            nmatch = _scalar(plsc.all_reduce_population_count(hit))
            plsc.store_compressed(scr_mv.at[:], mv, mask=hit)
            plsc.store_compressed(scr_mp.at[:], mp, mask=hit)

            def match_step(j, mc):
                jf = jnp.full((_LANES,), j, jnp.int32)
                lcol = plsc.load_gather(scr_mv.at[:], [jf]) - col0
                pos = _scalar(plsc.load_gather(scr_mp.at[:], [jf]))
                slot = mc & (_RING - 1)
                @pl.when(mc >= _RING)
                def _():
                    pltpu.make_async_copy(
                        out_hbm.at[0], rowbufs.at[0], sem.at[slot]
                    ).wait()
                for q in range(_EMBED_DIM // _LANES):
                    d_idx = _iota16() + q * _LANES
                    idxs = [lcol, d_idx] if swap else [d_idx, lcol]
                    g = plsc.load_gather(chunk_ref.at[:, :], idxs)
                    rowbufs[slot, pl.ds(q * _LANES, _LANES)] = g
                pltpu.async_copy(rowbufs.at[slot], out_hbm.at[pos], sem.at[slot])
                return mc + 1

            return lax.fori_loop(0, nmatch, match_step, mc)

        return lax.fori_loop(0, nvec, vec_step, mc)

    n_sweep = jnp.where(wid == 0, 62, 61)

    def start_load(k):
        c = k * _NUM_WORKERS + wid
        for h in range(4):
            quarter = _CHUNK // 4
            col0 = pl.multiple_of(c * _CHUNK + h * quarter, quarter)
            pltpu.async_copy(
                tableT_hbm.at[:, pl.ds(col0, quarter)],
                chunk_v.at[k & 1, :, pl.ds(h * quarter, quarter)],
                csem.at[k & 1],
            )

    start_load(0)

    def sweep_step(k, mc):
        @pl.when(k + 1 < n_sweep)
        def _():
            start_load(k + 1)

        pltpu.make_async_copy(
            tableT_hbm.at[:, pl.ds(0, _CHUNK)], chunk_v.at[k & 1],
            csem.at[k & 1],
        ).wait()
        c = k * _NUM_WORKERS + wid
        return process_chunk(c, c * _CHUNK, chunk_v.at[k & 1], mc)

    mc = lax.fori_loop(0, n_sweep, sweep_step, 0)

    def drain(mc):
        def wait_one(i, c):
            pltpu.make_async_copy(
                out_hbm.at[0], rowbufs.at[0], sem.at[i]
            ).wait()
            return c

        lax.fori_loop(0, jnp.minimum(mc, _RING), wait_one, 0)

    tail_owner = _N_FULL_CHUNKS & (_NUM_WORKERS - 1)

    @pl.when(wid == tail_owner)
    def _():
        pltpu.sync_copy(tail_hbm, tail_v)
        drain(process_chunk(_N_FULL_CHUNKS, _TAIL_COL0, tail_v, mc, swap=True))

    @pl.when(wid != tail_owner)
    def _():
        drain(mc)


def kernel(user_ids, table):
    mesh = plsc.VectorSubcoreMesh(core_axis_name="c", subcore_axis_name="s")
    f = pl.kernel(
        _gather_body,
        mesh=mesh,
        out_type=jax.ShapeDtypeStruct((_BATCH, _OUT_PAD), jnp.float32),
        scratch_types=[
            pltpu.VMEM((_BATCH + 2 * _LANES,), jnp.int32),
            pltpu.VMEM((_BATCH + 2 * _LANES,), jnp.int32),
            pltpu.VMEM((_BATCH + 2 * _LANES,), jnp.int32),
            pltpu.VMEM((2, _EMBED_DIM, _CHUNK), jnp.float32),
            pltpu.VMEM((_TAIL_W, _EMBED_DIM), jnp.float32),
            pltpu.VMEM((_LANES,), jnp.int32),
            pltpu.VMEM((_LANES,), jnp.int32),
            pltpu.VMEM((_RING, _OUT_PAD), jnp.float32),
            pltpu.SemaphoreType.DMA((_RING,)),
            pltpu.SemaphoreType.DMA((2,)),
        ],
        compiler_params=pltpu.CompilerParams(
            use_tc_tiling_on_sc=True, needs_layout_passes=False
        ),
    )
    tail_block = lax.slice(
        table, (_TAIL_COL0, 0), (_TAIL_COL0 + _TAIL_W, _EMBED_DIM)
    )
    out_pad = f(user_ids.astype(jnp.int32), table.T, tail_block)
    return out_pad[:, :_EMBED_DIM]


# speedup vs baseline: 1.0123x; 1.0123x over previous
import functools

import jax
import jax.numpy as jnp
from jax import lax
from jax.experimental import pallas as pl
from jax.experimental.pallas import tpu as pltpu
from jax.experimental.pallas import tpu_sc as plsc

_BATCH = 16384
_EMBED_DIM = 64
_OUT_PAD = 128
_NUM_CORES = 2
_NUM_SUBCORES = 16
_NUM_WORKERS = _NUM_CORES * _NUM_SUBCORES
_CHUNK = 512
_N_FULL_CHUNKS = 1953
_TAIL_COL0 = _N_FULL_CHUNKS * _CHUNK
_TAIL_W = 64
_RING = 16
_LANES = 16


def _iota16():
    return lax.broadcasted_iota(jnp.int32, (_LANES,), 0)


def _scalar(x):
    return x[0] if getattr(x, "ndim", 0) else x


def _gather_body(idx_hbm, tableT_hbm, tail_hbm, out_hbm, midx, mpos, gbuf,
                 chunk_v, tail_v, scr_mv, scr_mp, rowbufs, sem, csem):
    wid = lax.axis_index("s") * _NUM_CORES + lax.axis_index("c")
    pltpu.sync_copy(idx_hbm, gbuf.at[pl.ds(0, _BATCH)])

    def scan_step(t, cnt):
        v = gbuf[pl.ds(t * _LANES, _LANES)]
        pos = _iota16() + t * _LANES
        mine = ((v >> 9) & (_NUM_WORKERS - 1)) == wid
        plsc.store_compressed(midx.at[pl.ds(cnt, _LANES)], v, mask=mine)
        plsc.store_compressed(mpos.at[pl.ds(cnt, _LANES)], pos, mask=mine)
        return cnt + _scalar(plsc.all_reduce_population_count(mine))

    cnt = lax.fori_loop(0, _BATCH // _LANES, scan_step, 0)
    midx[pl.ds(cnt, _LANES)] = jnp.full((_LANES,), jnp.int32(0x7F000000))
    nvec = (cnt + _LANES - 1) // _LANES

    def process_chunk(c, col0, chunk_ref, mc, swap=False):
        def vec_step(t, mc):
            mv = midx[pl.ds(t * _LANES, _LANES)]
            mp = mpos[pl.ds(t * _LANES, _LANES)]
            hit = (mv >> 9) == c
            nmatch = _scalar(plsc.all_reduce_population_count(hit))
            plsc.store_compressed(scr_mv.at[:], mv, mask=hit)
            plsc.store_compressed(scr_mp.at[:], mp, mask=hit)

            def match_step(j, mc):
                jf = jnp.full((_LANES,), j, jnp.int32)
                lcol = plsc.load_gather(scr_mv.at[:], [jf]) - col0
                pos = _scalar(plsc.load_gather(scr_mp.at[:], [jf]))
                slot = mc & (_RING - 1)
                @pl.when(mc >= _RING)
                def _():
                    pltpu.make_async_copy(
                        out_hbm.at[0], rowbufs.at[0], sem.at[slot]
                    ).wait()
                for q in range(_EMBED_DIM // _LANES):
                    d_idx = _iota16() + q * _LANES
                    idxs = [lcol, d_idx] if swap else [d_idx, lcol]
                    g = plsc.load_gather(chunk_ref.at[:, :], idxs)
                    rowbufs[slot, pl.ds(q * _LANES, _LANES)] = g
                pltpu.async_copy(rowbufs.at[slot], out_hbm.at[pos], sem.at[slot])
                return mc + 1

            return lax.fori_loop(0, nmatch, match_step, mc)

        return lax.fori_loop(0, nvec, vec_step, mc)

    n_sweep = jnp.where(wid == 0, 62, 61)

    def start_load(k):
        c = k * _NUM_WORKERS + wid
        col0 = pl.multiple_of(c * _CHUNK, _CHUNK)
        pltpu.async_copy(
            tableT_hbm.at[:, pl.ds(col0, _CHUNK)],
            chunk_v.at[k & 1],
            csem.at[k & 1],
        )

    start_load(0)

    def sweep_step(k, mc):
        @pl.when(k + 1 < n_sweep)
        def _():
            start_load(k + 1)

        pltpu.make_async_copy(
            tableT_hbm.at[:, pl.ds(0, _CHUNK)], chunk_v.at[k & 1],
            csem.at[k & 1],
        ).wait()
        c = k * _NUM_WORKERS + wid
        return process_chunk(c, c * _CHUNK, chunk_v.at[k & 1], mc)

    mc = lax.fori_loop(0, n_sweep, sweep_step, 0)

    def drain(mc):
        def wait_one(i, c):
            pltpu.make_async_copy(
                out_hbm.at[0], rowbufs.at[0], sem.at[i]
            ).wait()
            return c

        lax.fori_loop(0, jnp.minimum(mc, _RING), wait_one, 0)

    tail_owner = _N_FULL_CHUNKS & (_NUM_WORKERS - 1)

    @pl.when(wid == tail_owner)
    def _():
        pltpu.sync_copy(tail_hbm, tail_v)
        drain(process_chunk(_N_FULL_CHUNKS, _TAIL_COL0, tail_v, mc, swap=True))

    @pl.when(wid != tail_owner)
    def _():
        drain(mc)


def kernel(user_ids, table):
    mesh = plsc.VectorSubcoreMesh(core_axis_name="c", subcore_axis_name="s")
    f = pl.kernel(
        _gather_body,
        mesh=mesh,
        out_type=jax.ShapeDtypeStruct((_BATCH, _OUT_PAD), jnp.float32),
        scratch_types=[
            pltpu.VMEM((_BATCH + 2 * _LANES,), jnp.int32),
            pltpu.VMEM((_BATCH + 2 * _LANES,), jnp.int32),
            pltpu.VMEM((_BATCH + 2 * _LANES,), jnp.int32),
            pltpu.VMEM((2, _EMBED_DIM, _CHUNK), jnp.float32),
            pltpu.VMEM((_TAIL_W, _EMBED_DIM), jnp.float32),
            pltpu.VMEM((_LANES,), jnp.int32),
            pltpu.VMEM((_LANES,), jnp.int32),
            pltpu.VMEM((_RING, _OUT_PAD), jnp.float32),
            pltpu.SemaphoreType.DMA((_RING,)),
            pltpu.SemaphoreType.DMA((2,)),
        ],
        compiler_params=pltpu.CompilerParams(
            use_tc_tiling_on_sc=True, needs_layout_passes=False
        ),
    )
    tail_block = lax.slice(
        table, (_TAIL_COL0, 0), (_TAIL_COL0 + _TAIL_W, _EMBED_DIM)
    )
    out_pad = f(user_ids.astype(jnp.int32), table.T, tail_block)
    return out_pad[:, :_EMBED_DIM]
